# baseline (speedup 1.0000x reference)
import jax
import jax.numpy as jnp
from jax.experimental import pallas as pl
from jax.experimental.pallas import tpu as pltpu

B, N, IN_C, HID, OUT_C = 16, 1024, 32, 32, 32
MTOT = float(B * N)
EPS = 1e-5

_WR = (0, 64, 128)
_WL = (32, 96, 160)
_WLIN = 192
_VROW = 288


def _body(x_ref, adj_ref, w_ref, out_ref,
          adjc, r_ref, s1, s2, a_ref, bb_ref):
    t = pl.program_id(0)
    bf16 = jnp.bfloat16

    @pl.when(t == 0)
    def _init_stats():
        s1[...] = jnp.zeros_like(s1)
        s2[...] = jnp.zeros_like(s2)

    @pl.when(jnp.logical_and(t >= 16, (t - 16) % 4 == 0))
    def _finalize_stats():
        j = (t - 16) // 4
        g = jnp.where(t == 16, w_ref[_VROW + 1:_VROW + 2],
                      jnp.where(t == 20, w_ref[_VROW + 4:_VROW + 5],
                                w_ref[_VROW + 7:_VROW + 8]))
        be = jnp.where(t == 16, w_ref[_VROW + 2:_VROW + 3],
                       jnp.where(t == 20, w_ref[_VROW + 5:_VROW + 6],
                                 w_ref[_VROW + 8:_VROW + 9]))
        mu = s1[...] / MTOT
        var = s2[...] / MTOT - mu * mu
        a = g * jax.lax.rsqrt(var + EPS)
        a_ref[j] = a
        bb_ref[j] = be - mu * a
        s1[...] = jnp.zeros_like(s1)
        s2[...] = jnp.zeros_like(s2)

    def layer(xin_bf, agg, lyr, jout, b):
        conv = (jnp.dot(agg.astype(bf16),
                        w_ref[_WR[lyr]:_WR[lyr] + 32].astype(bf16),
                        preferred_element_type=jnp.float32)
                + jnp.dot(xin_bf, w_ref[_WL[lyr]:_WL[lyr] + 32].astype(bf16),
                          preferred_element_type=jnp.float32)
                + w_ref[_VROW + 3 * lyr:_VROW + 3 * lyr + 1])
        r = jnp.maximum(conv, 0.0)
        r_ref[jout, b] = r.astype(bf16)
        s1[...] += jnp.sum(r, axis=0, keepdims=True)
        s2[...] += jnp.sum(r * r, axis=0, keepdims=True)

    def bn_apply(j, b):
        xf = a_ref[j] * r_ref[j, b].astype(jnp.float32) + bb_ref[j]
        return xf.astype(bf16)

    def agg_cached(b, x_bf):
        return sum(
            jnp.dot(adjc[b, :, 256 * c:256 * (c + 1)],
                    x_bf[256 * c:256 * (c + 1), :],
                    preferred_element_type=jnp.float32)
            for c in range(4))

    @pl.when(t < 16)
    def _phase0():
        ab = adj_ref[0].astype(bf16)
        adjc[t] = ab
        xb = x_ref[0].astype(bf16)
        agg = jnp.dot(ab, xb, preferred_element_type=jnp.float32)
        layer(xb, agg, 0, 0, t)

    @pl.when(jnp.logical_and(t >= 16, t < 20))
    def _phase1():
        for i in range(4):
            b = 4 * (t - 16) + i
            x1 = bn_apply(0, b)
            layer(x1, agg_cached(b, x1), 1, 1, b)

    @pl.when(jnp.logical_and(t >= 20, t < 24))
    def _phase2():
        for i in range(4):
            b = 4 * (t - 20) + i
            x2 = bn_apply(1, b)
            layer(x2, agg_cached(b, x2), 2, 2, b)

    @pl.when(t >= 24)
    def _phase3():
        wl1 = w_ref[_WLIN:_WLIN + 32].astype(bf16)
        wl2 = w_ref[_WLIN + 32:_WLIN + 64].astype(bf16)
        wl3 = w_ref[_WLIN + 64:_WLIN + 96].astype(bf16)
        blin = w_ref[_VROW + 9:_VROW + 10]
        for i in range(4):
            b = 4 * (t - 24) + i
            o = (jnp.dot(bn_apply(0, b), wl1, preferred_element_type=jnp.float32)
                 + jnp.dot(bn_apply(1, b), wl2, preferred_element_type=jnp.float32)
                 + jnp.dot(bn_apply(2, b), wl3, preferred_element_type=jnp.float32)
                 + blin)
            out_ref[i] = jnp.maximum(o, 0.0)


def kernel(x, adj, Wr1, br1, Wl1, g1, be1, Wr2, br2, Wl2, g2, be2,
           Wr3, br3, Wl3, g3, be3, Wlin, blin):
    vec = lambda v: v.reshape(1, -1)
    wpack = jnp.concatenate(
        [Wr1, Wl1, Wr2, Wl2, Wr3, Wl3, Wlin,
         vec(br1), vec(g1), vec(be1),
         vec(br2), vec(g2), vec(be2),
         vec(br3), vec(g3), vec(be3),
         vec(blin), jnp.zeros((6, HID), jnp.float32)], axis=0)

    return pl.pallas_call(
        _body,
        grid=(28,),
        in_specs=[
            pl.BlockSpec((1, N, IN_C),
                         lambda t: (jnp.where(t < 16, t, B - 1), 0, 0)),
            pl.BlockSpec((1, N, N),
                         lambda t: (jnp.where(t < 16, t, B - 1), 0, 0)),
            pl.BlockSpec(wpack.shape, lambda t: (0, 0)),
        ],
        out_specs=pl.BlockSpec(
            (4, N, OUT_C), lambda t: (jnp.where(t >= 24, t - 24, 0), 0, 0)),
        out_shape=jax.ShapeDtypeStruct((B, N, OUT_C), jnp.float32),
        scratch_shapes=[
            pltpu.VMEM((B, N, N), jnp.bfloat16),
            pltpu.VMEM((3, B, N, HID), jnp.bfloat16),
            pltpu.VMEM((1, HID), jnp.float32),
            pltpu.VMEM((1, HID), jnp.float32),
            pltpu.VMEM((3, 1, HID), jnp.float32),
            pltpu.VMEM((3, 1, HID), jnp.float32),
        ],
        compiler_params=pltpu.CompilerParams(
            dimension_semantics=("arbitrary",),
            vmem_limit_bytes=112 * 1024 * 1024,
        ),
    )(x, adj, wpack)


# speedup vs baseline: 1.0125x; 1.0125x over previous
import jax
import jax.numpy as jnp
from jax.experimental import pallas as pl
from jax.experimental.pallas import tpu as pltpu

B, N, IN_C, HID, OUT_C = 16, 1024, 32, 32, 32
MTOT = float(B * N)
EPS = 1e-5

_WR = (0, 64, 128)
_WL = (32, 96, 160)
_WLIN = 192
_VROW = 288


def _body(x_ref, adj_ref, w_ref, out_ref,
          adjc, r_ref, s1, s2, a_ref, bb_ref):
    t = pl.program_id(0)
    bf16 = jnp.bfloat16

    @pl.when(t == 0)
    def _init_stats():
        s1[...] = jnp.zeros_like(s1)
        s2[...] = jnp.zeros_like(s2)

    @pl.when(jnp.logical_and(t >= 16, t % 16 == 0))
    def _finalize_stats():
        j = t // 16 - 1
        g = jnp.where(t == 16, w_ref[_VROW + 1:_VROW + 2],
                      jnp.where(t == 32, w_ref[_VROW + 4:_VROW + 5],
                                w_ref[_VROW + 7:_VROW + 8]))
        be = jnp.where(t == 16, w_ref[_VROW + 2:_VROW + 3],
                       jnp.where(t == 32, w_ref[_VROW + 5:_VROW + 6],
                                 w_ref[_VROW + 8:_VROW + 9]))
        mu = s1[...] / MTOT
        var = s2[...] / MTOT - mu * mu
        a = g * jax.lax.rsqrt(var + EPS)
        a_ref[j] = a
        bb_ref[j] = be - mu * a
        s1[...] = jnp.zeros_like(s1)
        s2[...] = jnp.zeros_like(s2)

    def layer(xin_bf, agg, lyr, jout, b):
        conv = (jnp.dot(agg.astype(bf16),
                        w_ref[_WR[lyr]:_WR[lyr] + 32].astype(bf16),
                        preferred_element_type=jnp.float32)
                + jnp.dot(xin_bf, w_ref[_WL[lyr]:_WL[lyr] + 32].astype(bf16),
                          preferred_element_type=jnp.float32)
                + w_ref[_VROW + 3 * lyr:_VROW + 3 * lyr + 1])
        r = jnp.maximum(conv, 0.0)
        r_ref[jout, b] = r.astype(bf16)
        s1[...] += jnp.sum(r, axis=0, keepdims=True)
        s2[...] += jnp.sum(r * r, axis=0, keepdims=True)

    def bn_apply(j, b):
        xf = a_ref[j] * r_ref[j, b].astype(jnp.float32) + bb_ref[j]
        return xf.astype(bf16)

    def agg_cached(b, x_bf):
        return sum(
            jnp.dot(adjc[b, :, 256 * c:256 * (c + 1)],
                    x_bf[256 * c:256 * (c + 1), :],
                    preferred_element_type=jnp.float32)
            for c in range(4))

    @pl.when(t < 16)
    def _phase0():
        ab = adj_ref[0].astype(bf16)
        adjc[t] = ab
        xb = x_ref[0].astype(bf16)
        agg = jnp.dot(ab, xb, preferred_element_type=jnp.float32)
        layer(xb, agg, 0, 0, t)

    @pl.when(jnp.logical_and(t >= 16, t < 32))
    def _phase1():
        b = t - 16
        x1 = bn_apply(0, b)
        layer(x1, agg_cached(b, x1), 1, 1, b)

    @pl.when(jnp.logical_and(t >= 32, t < 48))
    def _phase2():
        b = t - 32
        x2 = bn_apply(1, b)
        layer(x2, agg_cached(b, x2), 2, 2, b)

    @pl.when(t >= 48)
    def _phase3():
        b = t - 48
        o = (jnp.dot(bn_apply(0, b), w_ref[_WLIN:_WLIN + 32].astype(bf16),
                     preferred_element_type=jnp.float32)
             + jnp.dot(bn_apply(1, b), w_ref[_WLIN + 32:_WLIN + 64].astype(bf16),
                       preferred_element_type=jnp.float32)
             + jnp.dot(bn_apply(2, b), w_ref[_WLIN + 64:_WLIN + 96].astype(bf16),
                       preferred_element_type=jnp.float32)
             + w_ref[_VROW + 9:_VROW + 10])
        out_ref[0] = jnp.maximum(o, 0.0)


def kernel(x, adj, Wr1, br1, Wl1, g1, be1, Wr2, br2, Wl2, g2, be2,
           Wr3, br3, Wl3, g3, be3, Wlin, blin):
    vec = lambda v: v.reshape(1, -1)
    wpack = jnp.concatenate(
        [Wr1, Wl1, Wr2, Wl2, Wr3, Wl3, Wlin,
         vec(br1), vec(g1), vec(be1),
         vec(br2), vec(g2), vec(be2),
         vec(br3), vec(g3), vec(be3),
         vec(blin), jnp.zeros((6, HID), jnp.float32)], axis=0)

    return pl.pallas_call(
        _body,
        grid=(64,),
        in_specs=[
            pl.BlockSpec((1, N, IN_C),
                         lambda t: (jnp.where(t < 16, t, B - 1), 0, 0)),
            pl.BlockSpec((1, N, N),
                         lambda t: (jnp.where(t < 16, t, B - 1), 0, 0)),
            pl.BlockSpec(wpack.shape, lambda t: (0, 0)),
        ],
        out_specs=pl.BlockSpec(
            (1, N, OUT_C), lambda t: (jnp.where(t >= 48, t - 48, 0), 0, 0)),
        out_shape=jax.ShapeDtypeStruct((B, N, OUT_C), jnp.float32),
        scratch_shapes=[
            pltpu.VMEM((B, N, N), jnp.bfloat16),
            pltpu.VMEM((3, B, N, HID), jnp.bfloat16),
            pltpu.VMEM((1, HID), jnp.float32),
            pltpu.VMEM((1, HID), jnp.float32),
            pltpu.VMEM((3, 1, HID), jnp.float32),
            pltpu.VMEM((3, 1, HID), jnp.float32),
        ],
        compiler_params=pltpu.CompilerParams(
            dimension_semantics=("arbitrary",),
            vmem_limit_bytes=112 * 1024 * 1024,
        ),
    )(x, adj, wpack)


# speedup vs baseline: 1.1934x; 1.1787x over previous
import jax
import jax.numpy as jnp
from jax.experimental import pallas as pl
from jax.experimental.pallas import tpu as pltpu

B, N, IN_C, HID, OUT_C = 16, 1024, 32, 32, 32
MTOT = float(B * N)
EPS = 1e-5


def _body(x_ref, adj_ref, wr1, br1, wl1, g1, be1, wr2, br2, wl2, g2, be2,
          wr3, br3, wl3, g3, be3, wlin, blin, out_ref,
          adjc, r_ref, s1, s2, a_ref, bb_ref):
    t = pl.program_id(0)
    bf16 = jnp.bfloat16

    @pl.when(t == 0)
    def _init_stats():
        s1[...] = jnp.zeros_like(s1)
        s2[...] = jnp.zeros_like(s2)

    @pl.when(jnp.logical_and(t >= 16, t <= 18))
    def _finalize_stats():
        j = t - 16
        g = jnp.where(t == 16, g1[...], jnp.where(t == 17, g2[...], g3[...]))
        be = jnp.where(t == 16, be1[...], jnp.where(t == 17, be2[...], be3[...]))
        mu = s1[...] / MTOT
        var = s2[...] / MTOT - mu * mu
        a = g * jax.lax.rsqrt(var + EPS)
        a_ref[j] = a
        bb_ref[j] = be - mu * a
        s1[...] = jnp.zeros_like(s1)
        s2[...] = jnp.zeros_like(s2)

    def layer(xin_bf, agg, wr, brv, wl, jout, b):
        conv = (jnp.dot(agg.astype(bf16), wr[...].astype(bf16),
                        preferred_element_type=jnp.float32)
                + jnp.dot(xin_bf, wl[...].astype(bf16),
                          preferred_element_type=jnp.float32)
                + brv[...])
        r = jnp.maximum(conv, 0.0)
        r_ref[jout, b] = r.astype(bf16)
        s1[...] += jnp.sum(r, axis=0, keepdims=True)
        s2[...] += jnp.sum(r * r, axis=0, keepdims=True)

    def bn_apply(j, b):
        xf = a_ref[j] * r_ref[j, b].astype(jnp.float32) + bb_ref[j]
        return xf.astype(bf16)

    def agg_cached(b, x_bf):
        return sum(
            jnp.dot(adjc[b, :, 256 * c:256 * (c + 1)],
                    x_bf[256 * c:256 * (c + 1), :],
                    preferred_element_type=jnp.float32)
            for c in range(4))

    @pl.when(t < 16)
    def _phase0():
        ab = adj_ref[0].astype(bf16)
        adjc[t] = ab
        xb = x_ref[0].astype(bf16)
        agg = jnp.dot(ab, xb, preferred_element_type=jnp.float32)
        layer(xb, agg, wr1, br1, wl1, 0, t)

    @pl.when(t == 16)
    def _phase1():
        def body1(b, carry):
            x1 = bn_apply(0, b)
            layer(x1, agg_cached(b, x1), wr2, br2, wl2, 1, b)
            return carry
        jax.lax.fori_loop(0, B, body1, 0)

    @pl.when(t == 17)
    def _phase2():
        def body2(b, carry):
            x2 = bn_apply(1, b)
            layer(x2, agg_cached(b, x2), wr3, br3, wl3, 2, b)
            return carry
        jax.lax.fori_loop(0, B, body2, 0)

    @pl.when(t >= 18)
    def _phase3():
        def body3(i, carry):
            b = 4 * (t - 18) + i
            o = (jnp.dot(bn_apply(0, b), wlin[0:HID].astype(bf16),
                         preferred_element_type=jnp.float32)
                 + jnp.dot(bn_apply(1, b), wlin[HID:2 * HID].astype(bf16),
                           preferred_element_type=jnp.float32)
                 + jnp.dot(bn_apply(2, b), wlin[2 * HID:].astype(bf16),
                           preferred_element_type=jnp.float32)
                 + blin[...])
            out_ref[i] = jnp.maximum(o, 0.0)
            return carry
        jax.lax.fori_loop(0, 4, body3, 0)


def kernel(x, adj, Wr1, br1, Wl1, g1, be1, Wr2, br2, Wl2, g2, be2,
           Wr3, br3, Wl3, g3, be3, Wlin, blin):
    vec = lambda v: v.reshape(1, -1)

    def full(arr):
        nd = arr.ndim
        return pl.BlockSpec(arr.shape, lambda t: (0,) * nd)

    small = [vec(br1), Wl1, vec(g1), vec(be1),
             Wr2, vec(br2), Wl2, vec(g2), vec(be2),
             Wr3, vec(br3), Wl3, vec(g3), vec(be3),
             Wlin, vec(blin)]

    in_specs = (
        [pl.BlockSpec((1, N, IN_C),
                      lambda t: (jnp.where(t < 16, t, B - 1), 0, 0)),
         pl.BlockSpec((1, N, N),
                      lambda t: (jnp.where(t < 16, t, B - 1), 0, 0)),
         full(Wr1)]
        + [full(a) for a in small]
    )

    return pl.pallas_call(
        _body,
        grid=(22,),
        in_specs=in_specs,
        out_specs=pl.BlockSpec(
            (4, N, OUT_C), lambda t: (jnp.where(t >= 18, t - 18, 0), 0, 0)),
        out_shape=jax.ShapeDtypeStruct((B, N, OUT_C), jnp.float32),
        scratch_shapes=[
            pltpu.VMEM((B, N, N), jnp.bfloat16),
            pltpu.VMEM((3, B, N, HID), jnp.bfloat16),
            pltpu.VMEM((1, HID), jnp.float32),
            pltpu.VMEM((1, HID), jnp.float32),
            pltpu.VMEM((3, 1, HID), jnp.float32),
            pltpu.VMEM((3, 1, HID), jnp.float32),
        ],
        compiler_params=pltpu.CompilerParams(
            dimension_semantics=("arbitrary",),
            vmem_limit_bytes=112 * 1024 * 1024,
        ),
    )(x, adj, Wr1, *small)
